# baseline (speedup 1.0000x reference)
import functools

import jax
import jax.numpy as jnp
from jax import lax
from jax.experimental import pallas as pl
from jax.experimental.pallas import tpu as pltpu
from jax.experimental.pallas import tpu_sc as plsc

_BC = 65536
_L = 128
_NF = _BC // _L


def _body(x_ref, lse_ref, mode_ref, m_ref, f_ref, s_ref,
          *, nsteps, ncols, bc):
    j = pl.program_id(0)
    B = m_ref.shape[0]

    @pl.when(j == 0)
    def _init():
        m_ref[...] = jnp.full_like(m_ref, -jnp.inf)
        f_ref[...] = jnp.zeros_like(f_ref)
        s_ref[...] = jnp.zeros_like(s_ref)

    lane = jax.lax.broadcasted_iota(jnp.int32, (B, _L), 1)

    def process(get_x):
        m_old = m_ref[...]
        m = m_old
        f = f_ref[...]
        for k in range(_NF):
            xk = get_x(k)
            c = xk > m
            m = jnp.where(c, xk, m)
            f = jnp.where(c, j * _NF + k, f)
        m_ref[...] = m
        f_ref[...] = f
        s_acc = jnp.zeros_like(m)
        for k in range(_NF):
            s_acc = s_acc + jnp.exp(get_x(k) - m)
        s_ref[...] = s_ref[...] * jnp.exp(m_old - m) + s_acc

    @pl.when(j < nsteps - 1)
    def _main():
        process(lambda k: x_ref[:, k * _L:(k + 1) * _L])

    @pl.when(j == nsteps - 1)
    def _last():
        lim = ncols - j * bc

        def get_x(k):
            xk = x_ref[:, k * _L:(k + 1) * _L]
            return jnp.where(lane + k * _L < lim, xk, -jnp.inf)

        process(get_x)

        m = m_ref[...]
        M = jnp.max(m, axis=1, keepdims=True)
        S = jnp.sum(s_ref[...] * jnp.exp(m - M), axis=1, keepdims=True)
        lse_ref[...] = M + jnp.log(S)
        cand = jnp.where(m == M, f_ref[...] * _L + lane, jnp.int32(2**30))
        mode_ref[...] = jnp.min(cand, axis=1, keepdims=True)


def _sc_gather_body(xf_hbm, a_hbm, out_hbm, a_v, idx_v, val_v, sem, *, ncols):
    cid = lax.axis_index("c")
    sid = lax.axis_index("s")

    @pl.when((cid == 0) & (sid == 0))
    def _():
        pltpu.sync_copy(a_hbm, a_v)
        for h in range(2):
            row = lax.broadcasted_iota(jnp.int32, (16,), 0) + 16 * h
            av = a_v[pl.ds(h * 16, 16)]
            idx_v[pl.ds(h * 16, 16)] = row * ncols + av
        pltpu.async_copy(xf_hbm.at[idx_v], val_v, sem).wait()
        pltpu.sync_copy(val_v, out_hbm)


@jax.jit
def kernel(logits, actions):
    B, C = logits.shape
    nsteps = pl.cdiv(C, _BC)

    lse, mode = pl.pallas_call(
        functools.partial(_body, nsteps=nsteps, ncols=C, bc=_BC),
        grid=(nsteps,),
        in_specs=[pl.BlockSpec((B, _BC), lambda j: (0, j))],
        out_specs=[
            pl.BlockSpec((B, 1), lambda j: (0, 0)),
            pl.BlockSpec((B, 1), lambda j: (0, 0)),
        ],
        out_shape=[
            jax.ShapeDtypeStruct((B, 1), jnp.float32),
            jax.ShapeDtypeStruct((B, 1), jnp.int32),
        ],
        scratch_shapes=[
            pltpu.VMEM((B, _L), jnp.float32),
            pltpu.VMEM((B, _L), jnp.int32),
            pltpu.VMEM((B, _L), jnp.float32),
        ],
    )(logits)

    sc_gather = pl.kernel(
        functools.partial(_sc_gather_body, ncols=C),
        out_type=jax.ShapeDtypeStruct((B,), jnp.float32),
        mesh=plsc.VectorSubcoreMesh(core_axis_name="c", subcore_axis_name="s"),
        scratch_types=[
            pltpu.VMEM((B,), jnp.int32),
            pltpu.VMEM((B,), jnp.int32),
            pltpu.VMEM((B,), jnp.float32),
            pltpu.SemaphoreType.DMA,
        ],
    )
    g = sc_gather(logits.reshape(B * C), actions.reshape(B))

    lp = g[:, None] - lse
    return lp, mode


# speedup vs baseline: 44.9107x; 44.9107x over previous
import functools

import jax
import jax.numpy as jnp
from jax.experimental import pallas as pl
from jax.experimental.pallas import tpu as pltpu

_BC = 65536
_L = 128
_NF = _BC // _L
_LOG2E = 1.4426950408889634


def _body(a_ref, x_ref, lp_ref, mode_ref, m_ref, f_ref, s_ref, g_ref,
          *, nsteps, ncols, bc):
    j = pl.program_id(0)
    B = m_ref.shape[0]

    @pl.when(j == 0)
    def _init():
        m_ref[...] = jnp.full_like(m_ref, -jnp.inf)
        f_ref[...] = jnp.zeros_like(f_ref)
        s_ref[...] = jnp.zeros_like(s_ref)
        g_ref[...] = jnp.zeros_like(g_ref)

    lane = jax.lax.broadcasted_iota(jnp.int32, (B, _L), 1)
    a = a_ref[...]

    def process(get_x):
        m_old = m_ref[...]
        m = m_old
        f = f_ref[...]
        g = g_ref[...]
        tcode = jnp.where(lane == a % _L, a // _L, jnp.int32(-1))
        tloc = tcode - j * _NF
        fl = jnp.full_like(f, -1)
        for k in range(_NF):
            xk = get_x(k)
            c = xk > m
            m = jnp.where(c, xk, m)
            fl = jnp.where(c, jnp.int32(k), fl)
            g = jnp.where(tloc == k, xk, g)
        m_ref[...] = m
        f_ref[...] = jnp.where(fl >= 0, fl + j * _NF, f)
        g_ref[...] = g
        s_acc = jnp.zeros_like(m)
        for k in range(_NF):
            s_acc = s_acc + jnp.exp(get_x(k) - m)
        s_ref[...] = s_ref[...] * jnp.exp(m_old - m) + s_acc

    @pl.when(j < nsteps - 1)
    def _main():
        process(lambda k: x_ref[:, k * _L:(k + 1) * _L])

    @pl.when(j == nsteps - 1)
    def _last():
        lim = ncols - j * bc

        def get_x(k):
            xk = x_ref[:, k * _L:(k + 1) * _L]
            return jnp.where(lane + k * _L < lim, xk, -jnp.inf)

        process(get_x)

        m = m_ref[...]
        M = jnp.max(m, axis=1, keepdims=True)
        S = jnp.sum(s_ref[...] * jnp.exp(m - M), axis=1, keepdims=True)
        lse = M + jnp.log(S)
        gval = jnp.sum(g_ref[...], axis=1, keepdims=True)
        lp_ref[...] = gval - lse
        cand = jnp.where(m == M, f_ref[...] * _L + lane, jnp.int32(2**30))
        mode_ref[...] = jnp.min(cand, axis=1, keepdims=True)


@jax.jit
def kernel(logits, actions):
    B, C = logits.shape
    nsteps = pl.cdiv(C, _BC)
    lp, mode = pl.pallas_call(
        functools.partial(_body, nsteps=nsteps, ncols=C, bc=_BC),
        grid=(nsteps,),
        in_specs=[
            pl.BlockSpec((B, 1), lambda j: (0, 0)),
            pl.BlockSpec((B, _BC), lambda j: (0, j)),
        ],
        out_specs=[
            pl.BlockSpec((B, 1), lambda j: (0, 0)),
            pl.BlockSpec((B, 1), lambda j: (0, 0)),
        ],
        out_shape=[
            jax.ShapeDtypeStruct((B, 1), jnp.float32),
            jax.ShapeDtypeStruct((B, 1), jnp.int32),
        ],
        scratch_shapes=[
            pltpu.VMEM((B, _L), jnp.float32),
            pltpu.VMEM((B, _L), jnp.int32),
            pltpu.VMEM((B, _L), jnp.float32),
            pltpu.VMEM((B, _L), jnp.float32),
        ],
    )(actions, logits)
    return lp, mode
